# baseline (speedup 1.0000x reference)
import functools

import jax
import jax.numpy as jnp
import numpy as np
from jax import lax
from jax.experimental import pallas as pl
from jax.experimental.pallas import tpu as pltpu
from jax.experimental.pallas import tpu_sc as plsc

N = 50000
E = 800000
K = 125
NTILES = 16
CHUNKS = E // NTILES // K
CB = 20
NBLK = CHUNKS // CB
NPAD = 50048
ROWS_PER_TILE = NPAD // NTILES
DRAIN = 136
NDRAIN = ROWS_PER_TILE // DRAIN
BM = 2000
GRID = N // BM


def _make_segsum(H):
    @functools.partial(
        pl.kernel,
        mesh=plsc.VectorSubcoreMesh(core_axis_name="c", subcore_axis_name="s"),
        compiler_params=pltpu.CompilerParams(use_tc_tiling_on_sc=False),
        out_type=[
            jax.ShapeDtypeStruct((NPAD, H), jnp.float32),
            jax.ShapeDtypeStruct((NPAD, H), jnp.float32),
        ],
    scratch_types=[
            pltpu.VMEM((2 * CB, K), jnp.int32),
            pltpu.VMEM((2 * CB, K), jnp.int32),
            pltpu.VMEM((K, H), jnp.float32),
            pltpu.VMEM((K, H), jnp.float32),
            pltpu.VMEM((K, H), jnp.float32),
            pltpu.VMEM((K, H), jnp.float32),
            pltpu.VMEM((K, H), jnp.float32),
            pltpu.VMEM_SHARED((NPAD, H), jnp.float32),
            pltpu.SemaphoreType.DMA,
            pltpu.SemaphoreType.DMA,
            pltpu.SemaphoreType.DMA,
            pltpu.SemaphoreType.DMA,
            pltpu.SemaphoreType.DMA,
            pltpu.SemaphoreType.DMA,
            pltpu.SemaphoreType.DMA,
            pltpu.SemaphoreType.DMA,
            pltpu.SemaphoreType.DMA,
            pltpu.SemaphoreType.DMA,
            pltpu.SemaphoreType.DMA,
            pltpu.SemaphoreType.DMA,
        ],
    )
    def _segsum(ylo, yhi, src2, dst2, zeros, outlo, outhi,
                sidx, didx, rows0, rows1, rows2, rows3, rows4, agg,
                g0, g1, g2, g3, g4, s0, s1, s2, s3, s4, semi0, semi1):
        c = lax.axis_index("c")
        s = lax.axis_index("s")
        slab = s * CHUNKS

        pltpu.sync_copy(zeros, agg.at[pl.ds(s * ROWS_PER_TILE, ROWS_PER_TILE)])
        pltpu.sync_copy(src2.at[pl.ds(slab, CB)], sidx.at[pl.ds(0, CB)])
        pltpu.sync_copy(dst2.at[pl.ds(slab, CB)], didx.at[pl.ds(0, CB)])
        plsc.subcore_barrier()

        def run(y_ref, out_ref):
            bufs = ((rows0, g0, s0), (rows1, g1, s1), (rows2, g2, s2),
                    (rows3, g3, s3), (rows4, g4, s4))

            def block(b, carry):
                p = lax.rem(b, 2)
                base = p * CB
                nxt = (1 - p) * CB
                first = b == 0

                @pl.when(b + 1 < NBLK)
                def _():
                    pltpu.async_copy(src2.at[pl.ds(slab + (b + 1) * CB, CB)],
                                     sidx.at[pl.ds(nxt, CB)], semi0)
                    pltpu.async_copy(dst2.at[pl.ds(slab + (b + 1) * CB, CB)],
                                     didx.at[pl.ds(nxt, CB)], semi1)

                def gather(j, buf, gsem):
                    pltpu.async_copy(y_ref.at[sidx.at[base + j]], buf, gsem)

                def gwait(j, buf, gsem):
                    pltpu.make_async_copy(
                        y_ref.at[sidx.at[base + j]], buf, gsem).wait()

                def scat(j, buf, ssem):
                    pltpu.async_copy(buf, agg.at[didx.at[base + j]], ssem,
                                     add=True)

                def swait(buf, ssem):
                    pltpu.make_async_copy(
                        buf, agg.at[didx.at[base]], ssem).wait()

                for gn in range(3):
                    buf, gs, ss = bufs[gn]

                    @pl.when(jnp.logical_not(first))
                    def _(buf=buf, ss=ss):
                        swait(buf, ss)

                    gather(gn, buf, gs)

                def quint(i, c2):
                    jb = i * 5
                    for u in range(5):
                        j = jb + u
                        buf, gs, ss = bufs[u]
                        gwait(j, buf, gs)
                        scat(j, buf, ss)
                        gn = j + 3
                        buf2, gs2, ss2 = bufs[(u + 3) % 5]

                        @pl.when(gn < CB)
                        def _(gn=gn, buf2=buf2, gs2=gs2, ss2=ss2):
                            @pl.when(jnp.logical_not(
                                jnp.logical_and(first, gn < 5)))
                            def _():
                                swait(buf2, ss2)
                            gather(gn, buf2, gs2)
                    return c2

                lax.fori_loop(0, CB // 5, quint, 0)

                @pl.when(b + 1 < NBLK)
                def _():
                    pltpu.make_async_copy(src2.at[pl.ds(slab, CB)],
                                          sidx.at[pl.ds(nxt, CB)], semi0).wait()
                    pltpu.make_async_copy(dst2.at[pl.ds(slab, CB)],
                                          didx.at[pl.ds(nxt, CB)], semi1).wait()

                return carry

            lax.fori_loop(0, NBLK, block, 0)

            for u in range(5):
                buf, gs, ss = bufs[u]
                pltpu.make_async_copy(buf, agg.at[didx.at[0]], ss).wait()

            plsc.subcore_barrier()

            r0 = s * ROWS_PER_TILE
            pltpu.sync_copy(agg.at[pl.ds(r0, ROWS_PER_TILE)],
                            out_ref.at[pl.ds(r0, ROWS_PER_TILE)])

        @pl.when(c == 0)
        def _():
            run(ylo, outlo)

        @pl.when(c != 0)
        def _():
            run(yhi, outhi)

    return _segsum


_segsum32 = _make_segsum(32)
_segsum8 = _make_segsum(8)


_BNS = np.float32(np.sqrt(np.float32(1.0 + 1e-5)))


def _mmbf(a, b):
    return jnp.dot(a.astype(jnp.bfloat16), b.astype(jnp.bfloat16),
                   preferred_element_type=jnp.float32)


def _gin_mlp(t, w1, b1, g, bt, w2, b2):
    u = _mmbf(t, w1) + b1
    u = u / _BNS * g + bt
    u = jnp.maximum(u, 0.0)
    return jnp.maximum(_mmbf(u, w2) + b2, 0.0)


def _tc1_body(x_ref, alo, ahi, atl, w1, b1, g, bt, w2, b2, hlo_ref, hhi_ref):
    agg = jnp.concatenate([alo[...], ahi[...], atl[...][:, :4]], axis=1)
    t = x_ref[...] + agg
    h = _gin_mlp(t, w1[...], b1[...], g[...], bt[...], w2[...], b2[...])
    hlo_ref[...] = h[:, :32]
    hhi_ref[...] = h[:, 32:]


def _tc1(x, alo, ahi, atl, w1, b1, g, bt, w2, b2):
    return pl.pallas_call(
        _tc1_body,
        grid=(GRID,),
        in_specs=[
            pl.BlockSpec((BM, 68), lambda i: (i, 0)),
            pl.BlockSpec((BM, 32), lambda i: (i, 0)),
            pl.BlockSpec((BM, 32), lambda i: (i, 0)),
            pl.BlockSpec((BM, 8), lambda i: (i, 0)),
            pl.BlockSpec((68, 64), lambda i: (0, 0)),
            pl.BlockSpec((1, 64), lambda i: (0, 0)),
            pl.BlockSpec((1, 64), lambda i: (0, 0)),
            pl.BlockSpec((1, 64), lambda i: (0, 0)),
            pl.BlockSpec((64, 64), lambda i: (0, 0)),
            pl.BlockSpec((1, 64), lambda i: (0, 0)),
        ],
        out_specs=[pl.BlockSpec((BM, 32), lambda i: (i, 0)),
                   pl.BlockSpec((BM, 32), lambda i: (i, 0))],
        out_shape=[jax.ShapeDtypeStruct((N, 32), jnp.float32),
                   jax.ShapeDtypeStruct((N, 32), jnp.float32)],
    )(x, alo, ahi, atl, w1, b1, g, bt, w2, b2)


def _tc2_body(hlo, hhi, alo, ahi, w1, b1, g, bt, w2, b2, olo_ref, ohi_ref):
    t = (jnp.concatenate([hlo[...], hhi[...]], axis=1)
         + jnp.concatenate([alo[...], ahi[...]], axis=1))
    h = _gin_mlp(t, w1[...], b1[...], g[...], bt[...], w2[...], b2[...])
    olo_ref[...] = h[:, :32]
    ohi_ref[...] = h[:, 32:]


def _tc2(hlo, hhi, alo, ahi, w1, b1, g, bt, w2, b2):
    half = pl.BlockSpec((BM, 32), lambda i: (i, 0))
    vec = pl.BlockSpec((1, 64), lambda i: (0, 0))
    return pl.pallas_call(
        _tc2_body,
        grid=(GRID,),
        in_specs=[half, half, half, half,
                  pl.BlockSpec((64, 64), lambda i: (0, 0)), vec, vec, vec,
                  pl.BlockSpec((64, 64), lambda i: (0, 0)), vec],
        out_specs=[half, half],
        out_shape=[jax.ShapeDtypeStruct((N, 32), jnp.float32),
                   jax.ShapeDtypeStruct((N, 32), jnp.float32)],
    )(hlo, hhi, alo, ahi, w1, b1, g, bt, w2, b2)


def _tcf_body(h1lo, h1hi, h2lo, h2hi, alo, ahi, w1, b1, g, bt, w2, b2,
              l1w, l1b, l2w, l2b, out_ref):
    h2 = jnp.concatenate([h2lo[...], h2hi[...]], axis=1)
    t = h2 + jnp.concatenate([alo[...], ahi[...]], axis=1)
    h3 = _gin_mlp(t, w1[...], b1[...], g[...], bt[...], w2[...], b2[...])
    hcat = jnp.concatenate([h1lo[...], h1hi[...], h2lo[...], h2hi[...], h3],
                           axis=1)
    z = jnp.maximum(_mmbf(hcat, l1w[...]) + l1b[...], 0.0)
    zb = z.astype(jnp.bfloat16).astype(jnp.float32)
    wb = l2w[...].astype(jnp.bfloat16).astype(jnp.float32)
    out_ref[...] = jnp.sum(zb * wb, axis=1, keepdims=True) + l2b[...]


def _tcf(h1lo, h1hi, h2lo, h2hi, alo, ahi, w1, b1, g, bt, w2, b2,
         l1w, l1b, l2w, l2b):
    half = pl.BlockSpec((BM, 32), lambda i: (i, 0))
    vec = pl.BlockSpec((1, 64), lambda i: (0, 0))
    return pl.pallas_call(
        _tcf_body,
        grid=(GRID,),
        in_specs=[half, half, half, half, half, half,
                  pl.BlockSpec((64, 64), lambda i: (0, 0)), vec, vec, vec,
                  pl.BlockSpec((64, 64), lambda i: (0, 0)), vec,
                  pl.BlockSpec((192, 192), lambda i: (0, 0)),
                  pl.BlockSpec((1, 192), lambda i: (0, 0)),
                  pl.BlockSpec((1, 192), lambda i: (0, 0)),
                  pl.BlockSpec((1, 1), lambda i: (0, 0))],
        out_specs=pl.BlockSpec((BM, 1), lambda i: (i, 0)),
        out_shape=jax.ShapeDtypeStruct((N, 1), jnp.float32),
    )(h1lo, h1hi, h2lo, h2hi, alo, ahi, w1, b1, g, bt, w2, b2,
      l1w, l1b, l2w, l2b)


def kernel(x, edge_index, c1_W1, c1_b1, c1_g, c1_bt, c1_W2, c1_b2,
           c2_W1, c2_b1, c2_g, c2_bt, c2_W2, c2_b2,
           c3_W1, c3_b1, c3_g, c3_bt, c3_W2, c3_b2,
           lin1_W, lin1_b, lin2_W, lin2_b):
    src2 = edge_index[0].reshape(E // K, K)
    dst2 = edge_index[1].reshape(E // K, K)
    z32 = jnp.zeros((ROWS_PER_TILE, 32), jnp.float32)
    z8 = jnp.zeros((ROWS_PER_TILE, 8), jnp.float32)
    xtail = jnp.pad(x[:, 64:], ((0, 0), (0, 4)))
    xzero = jnp.zeros((N, 8), jnp.float32)

    def r(v):
        return v.reshape(1, -1)

    a1lo, a1hi = _segsum32(x[:, :32], x[:, 32:64], src2, dst2, z32)
    a1tl, _unused = _segsum8(xtail, xzero, src2, dst2, z8)
    h1lo, h1hi = _tc1(x, a1lo, a1hi, a1tl, c1_W1,
                      r(c1_b1), r(c1_g), r(c1_bt), c1_W2, r(c1_b2))
    a2lo, a2hi = _segsum32(h1lo, h1hi, src2, dst2, z32)
    h2lo, h2hi = _tc2(h1lo, h1hi, a2lo, a2hi, c2_W1,
                      r(c2_b1), r(c2_g), r(c2_bt), c2_W2, r(c2_b2))
    a3lo, a3hi = _segsum32(h2lo, h2hi, src2, dst2, z32)
    out = _tcf(h1lo, h1hi, h2lo, h2hi, a3lo, a3hi, c3_W1,
               r(c3_b1), r(c3_g), r(c3_bt), c3_W2, r(c3_b2),
               lin1_W, r(lin1_b), lin2_W.T.reshape(1, 192),
               lin2_b.reshape(1, 1))
    return out


# speedup vs baseline: 1.0040x; 1.0040x over previous
import functools

import jax
import jax.numpy as jnp
import numpy as np
from jax import lax
from jax.experimental import pallas as pl
from jax.experimental.pallas import tpu as pltpu
from jax.experimental.pallas import tpu_sc as plsc

N = 50000
E = 800000
K = 125
NTILES = 16
CHUNKS = E // NTILES // K
CB = 16
NBLK = CHUNKS // CB
NPAD = 50048
ROWS_PER_TILE = NPAD // NTILES
DRAIN = 136
NDRAIN = ROWS_PER_TILE // DRAIN
BM = 2000
GRID = N // BM


def _make_segsum(H):
    @functools.partial(
        pl.kernel,
        mesh=plsc.VectorSubcoreMesh(core_axis_name="c", subcore_axis_name="s"),
        compiler_params=pltpu.CompilerParams(use_tc_tiling_on_sc=False),
        out_type=[
            jax.ShapeDtypeStruct((NPAD, H), jnp.float32),
            jax.ShapeDtypeStruct((NPAD, H), jnp.float32),
        ],
    scratch_types=[
            pltpu.VMEM((2 * CB, K), jnp.int32),
            pltpu.VMEM((2 * CB, K), jnp.int32),
            pltpu.VMEM((K, H), jnp.float32),
            pltpu.VMEM((K, H), jnp.float32),
            pltpu.VMEM((K, H), jnp.float32),
            pltpu.VMEM((K, H), jnp.float32),
            pltpu.VMEM_SHARED((NPAD, H), jnp.float32),
            pltpu.SemaphoreType.DMA,
            pltpu.SemaphoreType.DMA,
            pltpu.SemaphoreType.DMA,
            pltpu.SemaphoreType.DMA,
            pltpu.SemaphoreType.DMA,
            pltpu.SemaphoreType.DMA,
        ],
    )
    def _segsum(ylo, yhi, src2, dst2, zeros, outlo, outhi,
                sidx, didx, rows0, rows1, rows2, rows3, agg,
                sem0, sem1, sem2, sem3, semi0, semi1):
        c = lax.axis_index("c")
        s = lax.axis_index("s")
        slab = s * CHUNKS

        pltpu.sync_copy(zeros, agg.at[pl.ds(s * ROWS_PER_TILE, ROWS_PER_TILE)])
        pltpu.sync_copy(src2.at[pl.ds(slab, CB)], sidx.at[pl.ds(0, CB)])
        pltpu.sync_copy(dst2.at[pl.ds(slab, CB)], didx.at[pl.ds(0, CB)])
        plsc.subcore_barrier()

        def run(y_ref, out_ref):
            def block(b, carry):
                p = lax.rem(b, 2)
                base = p * CB
                nxt = (1 - p) * CB

                @pl.when(b + 1 < NBLK)
                def _():
                    pltpu.async_copy(src2.at[pl.ds(slab + (b + 1) * CB, CB)],
                                     sidx.at[pl.ds(nxt, CB)], semi0)
                    pltpu.async_copy(dst2.at[pl.ds(slab + (b + 1) * CB, CB)],
                                     didx.at[pl.ds(nxt, CB)], semi1)

                def gather(j, buf, sem):
                    pltpu.async_copy(y_ref.at[sidx.at[base + j]], buf, sem)

                def gwait(j, buf, sem):
                    pltpu.make_async_copy(
                        y_ref.at[sidx.at[base + j]], buf, sem).wait()

                def scat(j, buf):
                    pltpu.sync_copy(buf, agg.at[didx.at[base + j]], add=True)

                gather(0, rows0, sem0)
                gather(1, rows1, sem1)
                gather(2, rows2, sem2)

                def mbody(i, c2):
                    j = i * 4
                    gather(j + 3, rows3, sem3)
                    gwait(j, rows0, sem0)
                    scat(j, rows0)
                    gather(j + 4, rows0, sem0)
                    gwait(j + 1, rows1, sem1)
                    scat(j + 1, rows1)
                    gather(j + 5, rows1, sem1)
                    gwait(j + 2, rows2, sem2)
                    scat(j + 2, rows2)
                    gather(j + 6, rows2, sem2)
                    gwait(j + 3, rows3, sem3)
                    scat(j + 3, rows3)
                    return c2

                lax.fori_loop(0, CB // 4 - 1, mbody, 0)
                je = CB - 4
                gather(je + 3, rows3, sem3)
                gwait(je, rows0, sem0)
                scat(je, rows0)
                gwait(je + 1, rows1, sem1)
                scat(je + 1, rows1)
                gwait(je + 2, rows2, sem2)
                scat(je + 2, rows2)
                gwait(je + 3, rows3, sem3)
                scat(je + 3, rows3)

                @pl.when(b + 1 < NBLK)
                def _():
                    pltpu.make_async_copy(src2.at[pl.ds(slab, CB)],
                                          sidx.at[pl.ds(nxt, CB)], semi0).wait()
                    pltpu.make_async_copy(dst2.at[pl.ds(slab, CB)],
                                          didx.at[pl.ds(nxt, CB)], semi1).wait()

                return carry

            lax.fori_loop(0, NBLK, block, 0)

            plsc.subcore_barrier()

            r0 = s * ROWS_PER_TILE
            pltpu.sync_copy(agg.at[pl.ds(r0, ROWS_PER_TILE)],
                            out_ref.at[pl.ds(r0, ROWS_PER_TILE)])

        @pl.when(c == 0)
        def _():
            run(ylo, outlo)

        @pl.when(c != 0)
        def _():
            run(yhi, outhi)

    return _segsum


_segsum32 = _make_segsum(32)
_segsum8 = _make_segsum(8)


_BNS = np.float32(np.sqrt(np.float32(1.0 + 1e-5)))


def _mmbf(a, b):
    return jnp.dot(a.astype(jnp.bfloat16), b.astype(jnp.bfloat16),
                   preferred_element_type=jnp.float32)


def _gin_mlp(t, w1, b1, g, bt, w2, b2):
    u = _mmbf(t, w1) + b1
    u = u / _BNS * g + bt
    u = jnp.maximum(u, 0.0)
    return jnp.maximum(_mmbf(u, w2) + b2, 0.0)


def _tc1_body(x_ref, alo, ahi, atl, w1, b1, g, bt, w2, b2, hlo_ref, hhi_ref):
    agg = jnp.concatenate([alo[...], ahi[...], atl[...][:, :4]], axis=1)
    t = x_ref[...] + agg
    h = _gin_mlp(t, w1[...], b1[...], g[...], bt[...], w2[...], b2[...])
    hlo_ref[...] = h[:, :32]
    hhi_ref[...] = h[:, 32:]


def _tc1(x, alo, ahi, atl, w1, b1, g, bt, w2, b2):
    return pl.pallas_call(
        _tc1_body,
        grid=(GRID,),
        in_specs=[
            pl.BlockSpec((BM, 68), lambda i: (i, 0)),
            pl.BlockSpec((BM, 32), lambda i: (i, 0)),
            pl.BlockSpec((BM, 32), lambda i: (i, 0)),
            pl.BlockSpec((BM, 8), lambda i: (i, 0)),
            pl.BlockSpec((68, 64), lambda i: (0, 0)),
            pl.BlockSpec((1, 64), lambda i: (0, 0)),
            pl.BlockSpec((1, 64), lambda i: (0, 0)),
            pl.BlockSpec((1, 64), lambda i: (0, 0)),
            pl.BlockSpec((64, 64), lambda i: (0, 0)),
            pl.BlockSpec((1, 64), lambda i: (0, 0)),
        ],
        out_specs=[pl.BlockSpec((BM, 32), lambda i: (i, 0)),
                   pl.BlockSpec((BM, 32), lambda i: (i, 0))],
        out_shape=[jax.ShapeDtypeStruct((N, 32), jnp.float32),
                   jax.ShapeDtypeStruct((N, 32), jnp.float32)],
    )(x, alo, ahi, atl, w1, b1, g, bt, w2, b2)


def _tc2_body(hlo, hhi, alo, ahi, w1, b1, g, bt, w2, b2, olo_ref, ohi_ref):
    t = (jnp.concatenate([hlo[...], hhi[...]], axis=1)
         + jnp.concatenate([alo[...], ahi[...]], axis=1))
    h = _gin_mlp(t, w1[...], b1[...], g[...], bt[...], w2[...], b2[...])
    olo_ref[...] = h[:, :32]
    ohi_ref[...] = h[:, 32:]


def _tc2(hlo, hhi, alo, ahi, w1, b1, g, bt, w2, b2):
    half = pl.BlockSpec((BM, 32), lambda i: (i, 0))
    vec = pl.BlockSpec((1, 64), lambda i: (0, 0))
    return pl.pallas_call(
        _tc2_body,
        grid=(GRID,),
        in_specs=[half, half, half, half,
                  pl.BlockSpec((64, 64), lambda i: (0, 0)), vec, vec, vec,
                  pl.BlockSpec((64, 64), lambda i: (0, 0)), vec],
        out_specs=[half, half],
        out_shape=[jax.ShapeDtypeStruct((N, 32), jnp.float32),
                   jax.ShapeDtypeStruct((N, 32), jnp.float32)],
    )(hlo, hhi, alo, ahi, w1, b1, g, bt, w2, b2)


def _tcf_body(h1lo, h1hi, h2lo, h2hi, alo, ahi, w1, b1, g, bt, w2, b2,
              l1w, l1b, l2w, l2b, out_ref):
    h2 = jnp.concatenate([h2lo[...], h2hi[...]], axis=1)
    t = h2 + jnp.concatenate([alo[...], ahi[...]], axis=1)
    h3 = _gin_mlp(t, w1[...], b1[...], g[...], bt[...], w2[...], b2[...])
    hcat = jnp.concatenate([h1lo[...], h1hi[...], h2lo[...], h2hi[...], h3],
                           axis=1)
    z = jnp.maximum(_mmbf(hcat, l1w[...]) + l1b[...], 0.0)
    zb = z.astype(jnp.bfloat16).astype(jnp.float32)
    wb = l2w[...].astype(jnp.bfloat16).astype(jnp.float32)
    out_ref[...] = jnp.sum(zb * wb, axis=1, keepdims=True) + l2b[...]


def _tcf(h1lo, h1hi, h2lo, h2hi, alo, ahi, w1, b1, g, bt, w2, b2,
         l1w, l1b, l2w, l2b):
    half = pl.BlockSpec((BM, 32), lambda i: (i, 0))
    vec = pl.BlockSpec((1, 64), lambda i: (0, 0))
    return pl.pallas_call(
        _tcf_body,
        grid=(GRID,),
        in_specs=[half, half, half, half, half, half,
                  pl.BlockSpec((64, 64), lambda i: (0, 0)), vec, vec, vec,
                  pl.BlockSpec((64, 64), lambda i: (0, 0)), vec,
                  pl.BlockSpec((192, 192), lambda i: (0, 0)),
                  pl.BlockSpec((1, 192), lambda i: (0, 0)),
                  pl.BlockSpec((1, 192), lambda i: (0, 0)),
                  pl.BlockSpec((1, 1), lambda i: (0, 0))],
        out_specs=pl.BlockSpec((BM, 1), lambda i: (i, 0)),
        out_shape=jax.ShapeDtypeStruct((N, 1), jnp.float32),
    )(h1lo, h1hi, h2lo, h2hi, alo, ahi, w1, b1, g, bt, w2, b2,
      l1w, l1b, l2w, l2b)


def kernel(x, edge_index, c1_W1, c1_b1, c1_g, c1_bt, c1_W2, c1_b2,
           c2_W1, c2_b1, c2_g, c2_bt, c2_W2, c2_b2,
           c3_W1, c3_b1, c3_g, c3_bt, c3_W2, c3_b2,
           lin1_W, lin1_b, lin2_W, lin2_b):
    src2 = edge_index[0].reshape(E // K, K)
    dst2 = edge_index[1].reshape(E // K, K)
    z32 = jnp.zeros((ROWS_PER_TILE, 32), jnp.float32)
    z8 = jnp.zeros((ROWS_PER_TILE, 8), jnp.float32)
    xtail = jnp.pad(x[:, 64:], ((0, 0), (0, 4)))
    xzero = jnp.zeros((N, 8), jnp.float32)

    def r(v):
        return v.reshape(1, -1)

    a1lo, a1hi = _segsum32(x[:, :32], x[:, 32:64], src2, dst2, z32)
    a1tl, _unused = _segsum8(xtail, xzero, src2, dst2, z8)
    h1lo, h1hi = _tc1(x, a1lo, a1hi, a1tl, c1_W1,
                      r(c1_b1), r(c1_g), r(c1_bt), c1_W2, r(c1_b2))
    a2lo, a2hi = _segsum32(h1lo, h1hi, src2, dst2, z32)
    h2lo, h2hi = _tc2(h1lo, h1hi, a2lo, a2hi, c2_W1,
                      r(c2_b1), r(c2_g), r(c2_bt), c2_W2, r(c2_b2))
    a3lo, a3hi = _segsum32(h2lo, h2hi, src2, dst2, z32)
    out = _tcf(h1lo, h1hi, h2lo, h2hi, a3lo, a3hi, c3_W1,
               r(c3_b1), r(c3_g), r(c3_bt), c3_W2, r(c3_b2),
               lin1_W, r(lin1_b), lin2_W.T.reshape(1, 192),
               lin2_b.reshape(1, 1))
    return out
